# baseline (speedup 1.0000x reference)
import functools

import jax
import jax.numpy as jnp
from jax import lax
from jax.experimental import pallas as pl
from jax.experimental.pallas import tpu as pltpu
from jax.experimental.pallas import tpu_sc as plsc

B = 4096
NFEAT = 128
TDIM = 64
N0 = 10
N1 = 5
NCLASS = 100
NCPAD = 128

_NC = 2
_NS = 16
_NW = _NC * _NS
_IDX_TOTAL = B * N0
_IDX_PER_W = _IDX_TOTAL // _NW
_GATHERS_PER_W = _IDX_PER_W // 128
_HALF_G = _GATHERS_PER_W // 2
_ROWS_PER_ROUND = _HALF_G * 128
_OUT_PER_W = B // _NW


TNUM = 1000000
_CHUNK = 128
_NFULL_EVEN = 7808
_CPW = _NFULL_EVEN // _NW
_TAIL_E = TNUM - _NFULL_EVEN * _CHUNK
_PAIRS = TNUM // 2


_NBUF = 4


def _sc_transpose_body(tblT_hbm, tail_hbm, out_hbm, in_v, out_v, tail_v,
                       *sems):
  wid = lax.axis_index("s") * _NC + lax.axis_index("c")
  base = wid * _CPW
  sem_in = sems[:_NBUF]
  sem_out = sems[_NBUF:]

  def in_src(c):
    e0 = pl.multiple_of(c * _CHUNK, _CHUNK)
    return tblT_hbm.at[:, pl.ds(e0, _CHUNK)]

  def out_dst(c):
    r0 = pl.multiple_of(c * (_CHUNK // 2), _CHUNK // 2)
    return out_hbm.at[pl.ds(r0, _CHUNK // 2)]

  iota = lax.iota(jnp.int32, 16)
  tvecs = [iota + 16 * a for a in range(4)]
  em = [jnp.bitwise_and(iota + m, 15) for m in range(16)]
  srow = [lax.shift_right_logical(e, 1) for e in em]
  scolb = [jnp.bitwise_or(lax.shift_left(jnp.bitwise_and(e, 1), 6), iota)
           for e in em]

  for b in range(_NBUF):
    pltpu.async_copy(in_src(base + b), in_v.at[b], sem_in[b])

  def loop(nb, _):
    for b in range(_NBUF):
      c = base + _NBUF * nb + b
      pltpu.make_async_copy(in_src(c), in_v.at[b], sem_in[b]).wait()

      @pl.when(nb > 0)
      def _():
        pltpu.make_async_copy(out_v.at[b], out_dst(c), sem_out[b]).wait()

      src = in_v.at[b]
      dst = out_v.at[b]

      def eb_body(eb, _, src=src, dst=dst):
        for a in range(4):
          for m in range(16):
            ev = em[m] + 16 * eb
            rv = srow[m] + 8 * eb
            cv = scolb[m] + 16 * a
            v = plsc.load_gather(src, [tvecs[a], ev])
            plsc.store_scatter(dst, [rv, cv], v)
        return 0

      lax.fori_loop(0, 8, eb_body, 0)

      pltpu.async_copy(out_v.at[b], out_dst(c), sem_out[b])

      @pl.when(nb < _CPW // _NBUF - 1)
      def _():
        pltpu.async_copy(in_src(c + _NBUF), in_v.at[b], sem_in[b])
    return 0

  lax.fori_loop(0, _CPW // _NBUF, loop, 0)
  for b in range(_NBUF):
    pltpu.make_async_copy(out_v.at[b], out_dst(base), sem_out[b]).wait()

  @pl.when(wid == 0)
  def _():
    for t in range(_TAIL_E // 2 // 32):
      pltpu.sync_copy(tail_hbm.at[pl.ds(32 * t, 32)], tail_v)
      pltpu.sync_copy(
          tail_v,
          out_hbm.at[pl.ds(_NFULL_EVEN * (_CHUNK // 2) + 32 * t, 32)])


def _sc_pack(tableT, tail_pairs):
  mesh = plsc.VectorSubcoreMesh(core_axis_name="c", subcore_axis_name="s")
  return pl.kernel(
      _sc_transpose_body,
      out_type=jax.ShapeDtypeStruct((_PAIRS, 128), jnp.float32),
      mesh=mesh,
      scratch_types=[
          pltpu.VMEM((_NBUF, TDIM, _CHUNK), jnp.float32),
          pltpu.VMEM((_NBUF, _CHUNK // 2, 128), jnp.float32),
          pltpu.VMEM((32, 128), jnp.float32),
      ] + [pltpu.SemaphoreType.DMA] * (2 * _NBUF),
      compiler_params=pltpu.CompilerParams(needs_layout_passes=False),
  )(tableT, tail_pairs)


def _sc_gather_body(table_hbm, idx_hbm, out_hbm, idx_v, idxh_v, rows_v,
                    out_v, sem):
  wid = lax.axis_index("s") * _NC + lax.axis_index("c")
  pltpu.sync_copy(idx_hbm.at[wid], idx_v)
  for j in range(_GATHERS_PER_W):
    for c in range(128 // 16):
      sl = pl.ds(c * 16, 16)
      idxh_v[j, sl] = lax.shift_right_logical(idx_v[j, sl], 1)

  for half in range(2):
    copies = [
        pltpu.async_copy(table_hbm.at[idxh_v.at[half * _HALF_G + j]],
                         rows_v.at[pl.ds(j * 128, 128)], sem)
        for j in range(_HALF_G)
    ]
    for cp in copies:
      cp.wait()

    def group_mean(g, _, half=half):
      base = g * N0
      accs = [jnp.zeros((16,), jnp.float32) for _ in range(TDIM // 16)]
      for r in range(N0):
        row = base + r
        rv = jnp.full((16,), row, jnp.int32)
        jv = lax.shift_right_logical(rv, 7)
        kv = jnp.bitwise_and(rv, 127)
        orig = plsc.load_gather(
            idx_v, [jv + jnp.int32(half * _HALF_G), kv])
        odd = jnp.bitwise_and(orig, 1) == 1
        for c in range(TDIM // 16):
          lo = rows_v[row, pl.ds(c * 16, 16)]
          hi = rows_v[row, pl.ds(TDIM + c * 16, 16)]
          accs[c] = accs[c] + jnp.where(odd, hi, lo)
      for c in range(TDIM // 16):
        out_v[half * (_OUT_PER_W // 2) + g, pl.ds(c * 16, 16)] = (
            accs[c] * (1.0 / N0))
      return 0

    lax.fori_loop(0, _ROWS_PER_ROUND // N0, group_mean, 0)

  pltpu.sync_copy(out_v, out_hbm.at[pl.ds(wid * _OUT_PER_W, _OUT_PER_W)])


def _sc_gather_mean(pair_table, idx3d):
  mesh = plsc.VectorSubcoreMesh(core_axis_name="c", subcore_axis_name="s")
  return pl.kernel(
      _sc_gather_body,
      out_type=jax.ShapeDtypeStruct((B, TDIM), jnp.float32),
      mesh=mesh,
      scratch_types=[
          pltpu.VMEM((_GATHERS_PER_W, 128), jnp.int32),
          pltpu.VMEM((_GATHERS_PER_W, 128), jnp.int32),
          pltpu.VMEM((_ROWS_PER_ROUND, 128), jnp.float32),
          pltpu.VMEM((_OUT_PER_W, TDIM), jnp.float32),
          pltpu.SemaphoreType.DMA,
      ],
      compiler_params=pltpu.CompilerParams(needs_layout_passes=False),
  )(pair_table, idx3d)


_BB = 128


def _tca_body(x0_ref, x1_ref, x2_ref, xb2_ref, w_ref, a10_ref, a50_ref,
              e0_ref, e1_ref, e2_ref, b2_ref):
  w = w_ref[...]
  e0_ref[...] = lax.dot(x0_ref[...], w, preferred_element_type=jnp.float32)
  y1 = lax.dot(x1_ref[...], w, preferred_element_type=jnp.float32)
  e1_ref[...] = lax.dot(a10_ref[...], y1,
                        preferred_element_type=jnp.float32) * (1.0 / N0)
  y2 = lax.dot(x2_ref[...], w, preferred_element_type=jnp.float32)
  e2_ref[...] = lax.dot(a50_ref[...], y2,
                        preferred_element_type=jnp.float32) * (1.0 / (N0 * N1))
  yb = lax.dot(xb2_ref[...], w, preferred_element_type=jnp.float32)
  b2_ref[...] = lax.dot(a50_ref[...], yb,
                        preferred_element_type=jnp.float32) * (1.0 / (N0 * N1))


def _tc_a(x_0, x_1, x_2, x_bi_2, weight_trans, a10, a50):
  grid = (B // _BB,)
  out = jax.ShapeDtypeStruct((B, TDIM), jnp.float32)
  return pl.pallas_call(
      _tca_body,
      grid=grid,
      in_specs=[
          pl.BlockSpec((_BB, NFEAT), lambda i: (i, 0)),
          pl.BlockSpec((_BB * N0, NFEAT), lambda i: (i, 0)),
          pl.BlockSpec((_BB * N0 * N1, NFEAT), lambda i: (i, 0)),
          pl.BlockSpec((_BB * N0 * N1, NFEAT), lambda i: (i, 0)),
          pl.BlockSpec((NFEAT, TDIM), lambda i: (0, 0)),
          pl.BlockSpec((_BB, _BB * N0), lambda i: (0, 0)),
          pl.BlockSpec((_BB, _BB * N0 * N1), lambda i: (0, 0)),
      ],
      out_specs=[pl.BlockSpec((_BB, TDIM), lambda i: (i, 0))] * 4,
      out_shape=[out, out, out, out],
  )(x_0, x_1, x_2, x_bi_2, weight_trans, a10, a50)


_BBB = 1024


def _tcb_body(e0_ref, e1_ref, e2_ref, b2_ref, b1_ref, w1_ref, l2wt_ref,
              l2b_ref, out_ref):
  e0, e1, e2 = e0_ref[...], e1_ref[...], e2_ref[...]
  b1, b2 = b1_ref[...], b2_ref[...]
  terms = [e0 * b1, e0 * b2, e1 * b1, e1 * b2, e2 * b1, e2 * b2,
           e0 * e1, e0 * e2, e1 * e2, e0, e1, e2]
  w1 = w1_ref[...]
  logits = [jnp.sum(t * w1, axis=1, keepdims=True) for t in terms]
  mx = logits[0]
  for l in logits[1:]:
    mx = jnp.maximum(mx, l)
  exps = [jnp.exp(l - mx) for l in logits]
  denom = exps[0]
  for e in exps[1:]:
    denom = denom + e
  hidden = exps[0] * terms[0]
  for e, t in zip(exps[1:], terms[1:]):
    hidden = hidden + e * t
  hidden = hidden / denom

  out = lax.dot(hidden, l2wt_ref[...],
                preferred_element_type=jnp.float32) + l2b_ref[...]
  col = lax.broadcasted_iota(jnp.int32, out.shape, 1)
  out = jnp.where(col < NCLASS, out, -1e30)
  omx = jnp.max(out, axis=1, keepdims=True)
  sh = out - omx
  lse = jnp.log(jnp.sum(jnp.exp(sh), axis=1, keepdims=True))
  out_ref[...] = sh - lse


def _tc_b(e0, e1, e2, b2, b1, lin1_w, l2wt, l2b):
  grid = (B // _BBB,)
  return pl.pallas_call(
      _tcb_body,
      grid=grid,
      in_specs=[pl.BlockSpec((_BBB, TDIM), lambda i: (i, 0))] * 5 + [
          pl.BlockSpec((1, TDIM), lambda i: (0, 0)),
          pl.BlockSpec((TDIM, NCPAD), lambda i: (0, 0)),
          pl.BlockSpec((1, NCPAD), lambda i: (0, 0)),
      ],
      out_specs=pl.BlockSpec((_BBB, NCPAD), lambda i: (i, 0)),
      out_shape=jax.ShapeDtypeStruct((B, NCPAD), jnp.float32),
  )(e0, e1, e2, b2, b1, lin1_w, l2wt, l2b)


def kernel(x_0, x_1, x_2, x_bi_0, x_bi_1, x_bi_2, weight_trans, table,
           lin1_w, lin1_b, lin2_w, lin2_b):
  del x_bi_0, lin1_b
  idx3d = x_bi_1.astype(jnp.int32).reshape(_NW, _GATHERS_PER_W, 128)
  tail_pairs = table[_NFULL_EVEN * _CHUNK:].reshape(_TAIL_E // 2, 128)
  pair_table = _sc_pack(table.T, tail_pairs)
  b1 = _sc_gather_mean(pair_table, idx3d)

  ri10 = lax.broadcasted_iota(jnp.int32, (_BB, _BB * N0), 1) // N0
  a10 = (ri10 == lax.broadcasted_iota(jnp.int32, (_BB, _BB * N0), 0)
         ).astype(jnp.float32)
  ri50 = lax.broadcasted_iota(jnp.int32, (_BB, _BB * N0 * N1), 1) // (N0 * N1)
  a50 = (ri50 == lax.broadcasted_iota(jnp.int32, (_BB, _BB * N0 * N1), 0)
         ).astype(jnp.float32)

  e0, e1, e2, b2 = _tc_a(x_0, x_1, x_2, x_bi_2, weight_trans, a10, a50)

  l2wt = jnp.zeros((TDIM, NCPAD), jnp.float32).at[:, :NCLASS].set(lin2_w.T)
  l2b = jnp.zeros((1, NCPAD), jnp.float32).at[:, :NCLASS].set(lin2_b)
  out = _tc_b(e0, e1, e2, b2, b1, lin1_w, l2wt, l2b)
  return out[:, :NCLASS]


# speedup vs baseline: 2.0862x; 2.0862x over previous
import functools

import jax
import jax.numpy as jnp
from jax import lax
from jax.experimental import pallas as pl
from jax.experimental.pallas import tpu as pltpu
from jax.experimental.pallas import tpu_sc as plsc

B = 4096
NFEAT = 128
TDIM = 64
N0 = 10
N1 = 5
NCLASS = 100
NCPAD = 128

_NC = 2
_NS = 16
_NW = _NC * _NS
_IDX_TOTAL = B * N0
_IDX_PER_W = _IDX_TOTAL // _NW
_GATHERS_PER_W = _IDX_PER_W // 128
_HALF_G = _GATHERS_PER_W // 2
_ROWS_PER_ROUND = _HALF_G * 128
_OUT_PER_W = B // _NW


TNUM = 1000000
_CHUNK = 128
_NFULL_EVEN = 7808
_CPW = _NFULL_EVEN // _NW
_TAIL_E = TNUM - _NFULL_EVEN * _CHUNK
_PAIRS = TNUM // 2


_NBUF = 4


def _sc_transpose_body(tblT_hbm, tail_hbm, out_hbm, in_v, out_v, tail_v,
                       *sems):
  wid = lax.axis_index("s") * _NC + lax.axis_index("c")
  base = wid * _CPW
  sem_in = sems[:_NBUF]
  sem_out = sems[_NBUF:]

  def in_src(c):
    e0 = pl.multiple_of(c * _CHUNK, _CHUNK)
    return tblT_hbm.at[:, pl.ds(e0, _CHUNK)]

  def out_dst(c):
    r0 = pl.multiple_of(c * (_CHUNK // 2), _CHUNK // 2)
    return out_hbm.at[pl.ds(r0, _CHUNK // 2)]

  iota = lax.iota(jnp.int32, 16)
  tvecs = [iota + 16 * a for a in range(4)]
  em = [jnp.bitwise_and(iota + m, 15) for m in range(16)]
  srow = [lax.shift_right_logical(e, 1) for e in em]
  scolb = [jnp.bitwise_or(lax.shift_left(jnp.bitwise_and(e, 1), 6), iota)
           for e in em]

  for b in range(_NBUF):
    pltpu.async_copy(in_src(base + b), in_v.at[b], sem_in[b])

  def loop(nb, _):
    for b in range(_NBUF):
      c = base + _NBUF * nb + b
      pltpu.make_async_copy(in_src(c), in_v.at[b], sem_in[b]).wait()

      @pl.when(nb > 0)
      def _():
        pltpu.make_async_copy(out_v.at[b], out_dst(c), sem_out[b]).wait()

      src = in_v.at[b]
      dst = out_v.at[b]

      def eb_body(eb, _, src=src, dst=dst):
        for a in range(4):
          vals = []
          for m in range(16):
            ev = em[m] + 16 * eb
            vals.append(plsc.load_gather(src, [tvecs[a], ev]))
          for m in range(16):
            rv = srow[m] + 8 * eb
            cv = scolb[m] + 16 * a
            plsc.store_scatter(dst, [rv, cv], vals[m])
        return 0

      lax.fori_loop(0, 8, eb_body, 0)

      pltpu.async_copy(out_v.at[b], out_dst(c), sem_out[b])

      @pl.when(nb < _CPW // _NBUF - 1)
      def _():
        pltpu.async_copy(in_src(c + _NBUF), in_v.at[b], sem_in[b])
    return 0

  lax.fori_loop(0, _CPW // _NBUF, loop, 0)
  for b in range(_NBUF):
    pltpu.make_async_copy(out_v.at[b], out_dst(base), sem_out[b]).wait()

  @pl.when(wid == 0)
  def _():
    for t in range(_TAIL_E // 2 // 32):
      pltpu.sync_copy(tail_hbm.at[pl.ds(32 * t, 32)], tail_v)
      pltpu.sync_copy(
          tail_v,
          out_hbm.at[pl.ds(_NFULL_EVEN * (_CHUNK // 2) + 32 * t, 32)])


def _sc_pack(tableT, tail_pairs):
  mesh = plsc.VectorSubcoreMesh(core_axis_name="c", subcore_axis_name="s")
  return pl.kernel(
      _sc_transpose_body,
      out_type=jax.ShapeDtypeStruct((_PAIRS, 128), jnp.float32),
      mesh=mesh,
      scratch_types=[
          pltpu.VMEM((_NBUF, TDIM, _CHUNK), jnp.float32),
          pltpu.VMEM((_NBUF, _CHUNK // 2, 128), jnp.float32),
          pltpu.VMEM((32, 128), jnp.float32),
      ] + [pltpu.SemaphoreType.DMA] * (2 * _NBUF),
      compiler_params=pltpu.CompilerParams(needs_layout_passes=False),
  )(tableT, tail_pairs)


def _sc_gather_body(table_hbm, idx_hbm, out_hbm, idx_v, idxh_v, rows_v,
                    out_v, sem):
  wid = lax.axis_index("s") * _NC + lax.axis_index("c")
  pltpu.sync_copy(idx_hbm.at[wid], idx_v)
  for j in range(_GATHERS_PER_W):
    for c in range(128 // 16):
      sl = pl.ds(c * 16, 16)
      idxh_v[j, sl] = lax.shift_right_logical(idx_v[j, sl], 1)

  for half in range(2):
    copies = [
        pltpu.async_copy(table_hbm.at[idxh_v.at[half * _HALF_G + j]],
                         rows_v.at[pl.ds(j * 128, 128)], sem)
        for j in range(_HALF_G)
    ]
    for cp in copies:
      cp.wait()

    def group_mean(g, _, half=half):
      base = g * N0
      accs = [jnp.zeros((16,), jnp.float32) for _ in range(TDIM // 16)]
      for r in range(N0):
        row = base + r
        rv = jnp.full((16,), row, jnp.int32)
        jv = lax.shift_right_logical(rv, 7)
        kv = jnp.bitwise_and(rv, 127)
        orig = plsc.load_gather(
            idx_v, [jv + jnp.int32(half * _HALF_G), kv])
        odd = jnp.bitwise_and(orig, 1) == 1
        for c in range(TDIM // 16):
          lo = rows_v[row, pl.ds(c * 16, 16)]
          hi = rows_v[row, pl.ds(TDIM + c * 16, 16)]
          accs[c] = accs[c] + jnp.where(odd, hi, lo)
      for c in range(TDIM // 16):
        out_v[half * (_OUT_PER_W // 2) + g, pl.ds(c * 16, 16)] = (
            accs[c] * (1.0 / N0))
      return 0

    lax.fori_loop(0, _ROWS_PER_ROUND // N0, group_mean, 0)

  pltpu.sync_copy(out_v, out_hbm.at[pl.ds(wid * _OUT_PER_W, _OUT_PER_W)])


def _sc_gather_mean(pair_table, idx3d):
  mesh = plsc.VectorSubcoreMesh(core_axis_name="c", subcore_axis_name="s")
  return pl.kernel(
      _sc_gather_body,
      out_type=jax.ShapeDtypeStruct((B, TDIM), jnp.float32),
      mesh=mesh,
      scratch_types=[
          pltpu.VMEM((_GATHERS_PER_W, 128), jnp.int32),
          pltpu.VMEM((_GATHERS_PER_W, 128), jnp.int32),
          pltpu.VMEM((_ROWS_PER_ROUND, 128), jnp.float32),
          pltpu.VMEM((_OUT_PER_W, TDIM), jnp.float32),
          pltpu.SemaphoreType.DMA,
      ],
      compiler_params=pltpu.CompilerParams(needs_layout_passes=False),
  )(pair_table, idx3d)


_BB = 128


def _tca_body(x0_ref, x1_ref, x2_ref, xb2_ref, w_ref, a10_ref, a50_ref,
              e0_ref, e1_ref, e2_ref, b2_ref):
  w = w_ref[...]
  e0_ref[...] = lax.dot(x0_ref[...], w, preferred_element_type=jnp.float32)
  y1 = lax.dot(x1_ref[...], w, preferred_element_type=jnp.float32)
  e1_ref[...] = lax.dot(a10_ref[...], y1,
                        preferred_element_type=jnp.float32) * (1.0 / N0)
  y2 = lax.dot(x2_ref[...], w, preferred_element_type=jnp.float32)
  e2_ref[...] = lax.dot(a50_ref[...], y2,
                        preferred_element_type=jnp.float32) * (1.0 / (N0 * N1))
  yb = lax.dot(xb2_ref[...], w, preferred_element_type=jnp.float32)
  b2_ref[...] = lax.dot(a50_ref[...], yb,
                        preferred_element_type=jnp.float32) * (1.0 / (N0 * N1))


def _tc_a(x_0, x_1, x_2, x_bi_2, weight_trans, a10, a50):
  grid = (B // _BB,)
  out = jax.ShapeDtypeStruct((B, TDIM), jnp.float32)
  return pl.pallas_call(
      _tca_body,
      grid=grid,
      in_specs=[
          pl.BlockSpec((_BB, NFEAT), lambda i: (i, 0)),
          pl.BlockSpec((_BB * N0, NFEAT), lambda i: (i, 0)),
          pl.BlockSpec((_BB * N0 * N1, NFEAT), lambda i: (i, 0)),
          pl.BlockSpec((_BB * N0 * N1, NFEAT), lambda i: (i, 0)),
          pl.BlockSpec((NFEAT, TDIM), lambda i: (0, 0)),
          pl.BlockSpec((_BB, _BB * N0), lambda i: (0, 0)),
          pl.BlockSpec((_BB, _BB * N0 * N1), lambda i: (0, 0)),
      ],
      out_specs=[pl.BlockSpec((_BB, TDIM), lambda i: (i, 0))] * 4,
      out_shape=[out, out, out, out],
  )(x_0, x_1, x_2, x_bi_2, weight_trans, a10, a50)


_BBB = 1024


def _tcb_body(e0_ref, e1_ref, e2_ref, b2_ref, b1_ref, w1_ref, l2wt_ref,
              l2b_ref, out_ref):
  e0, e1, e2 = e0_ref[...], e1_ref[...], e2_ref[...]
  b1, b2 = b1_ref[...], b2_ref[...]
  terms = [e0 * b1, e0 * b2, e1 * b1, e1 * b2, e2 * b1, e2 * b2,
           e0 * e1, e0 * e2, e1 * e2, e0, e1, e2]
  w1 = w1_ref[...]
  logits = [jnp.sum(t * w1, axis=1, keepdims=True) for t in terms]
  mx = logits[0]
  for l in logits[1:]:
    mx = jnp.maximum(mx, l)
  exps = [jnp.exp(l - mx) for l in logits]
  denom = exps[0]
  for e in exps[1:]:
    denom = denom + e
  hidden = exps[0] * terms[0]
  for e, t in zip(exps[1:], terms[1:]):
    hidden = hidden + e * t
  hidden = hidden / denom

  out = lax.dot(hidden, l2wt_ref[...],
                preferred_element_type=jnp.float32) + l2b_ref[...]
  col = lax.broadcasted_iota(jnp.int32, out.shape, 1)
  out = jnp.where(col < NCLASS, out, -1e30)
  omx = jnp.max(out, axis=1, keepdims=True)
  sh = out - omx
  lse = jnp.log(jnp.sum(jnp.exp(sh), axis=1, keepdims=True))
  out_ref[...] = sh - lse


def _tc_b(e0, e1, e2, b2, b1, lin1_w, l2wt, l2b):
  grid = (B // _BBB,)
  return pl.pallas_call(
      _tcb_body,
      grid=grid,
      in_specs=[pl.BlockSpec((_BBB, TDIM), lambda i: (i, 0))] * 5 + [
          pl.BlockSpec((1, TDIM), lambda i: (0, 0)),
          pl.BlockSpec((TDIM, NCPAD), lambda i: (0, 0)),
          pl.BlockSpec((1, NCPAD), lambda i: (0, 0)),
      ],
      out_specs=pl.BlockSpec((_BBB, NCPAD), lambda i: (i, 0)),
      out_shape=jax.ShapeDtypeStruct((B, NCPAD), jnp.float32),
  )(e0, e1, e2, b2, b1, lin1_w, l2wt, l2b)


def kernel(x_0, x_1, x_2, x_bi_0, x_bi_1, x_bi_2, weight_trans, table,
           lin1_w, lin1_b, lin2_w, lin2_b):
  del x_bi_0, lin1_b
  idx3d = x_bi_1.astype(jnp.int32).reshape(_NW, _GATHERS_PER_W, 128)
  tail_pairs = table[_NFULL_EVEN * _CHUNK:].reshape(_TAIL_E // 2, 128)
  pair_table = _sc_pack(table.T, tail_pairs)
  b1 = _sc_gather_mean(pair_table, idx3d)

  ri10 = lax.broadcasted_iota(jnp.int32, (_BB, _BB * N0), 1) // N0
  a10 = (ri10 == lax.broadcasted_iota(jnp.int32, (_BB, _BB * N0), 0)
         ).astype(jnp.float32)
  ri50 = lax.broadcasted_iota(jnp.int32, (_BB, _BB * N0 * N1), 1) // (N0 * N1)
  a50 = (ri50 == lax.broadcasted_iota(jnp.int32, (_BB, _BB * N0 * N1), 0)
         ).astype(jnp.float32)

  e0, e1, e2, b2 = _tc_a(x_0, x_1, x_2, x_bi_2, weight_trans, a10, a50)

  l2wt = jnp.zeros((TDIM, NCPAD), jnp.float32).at[:, :NCLASS].set(lin2_w.T)
  l2b = jnp.zeros((1, NCPAD), jnp.float32).at[:, :NCLASS].set(lin2_b)
  out = _tc_b(e0, e1, e2, b2, b1, lin1_w, l2wt, l2b)
  return out[:, :NCLASS]


# speedup vs baseline: 2.0864x; 1.0001x over previous
import functools

import jax
import jax.numpy as jnp
from jax import lax
from jax.experimental import pallas as pl
from jax.experimental.pallas import tpu as pltpu
from jax.experimental.pallas import tpu_sc as plsc

B = 4096
NFEAT = 128
TDIM = 64
N0 = 10
N1 = 5
NCLASS = 100
NCPAD = 128

_NC = 2
_NS = 16
_NW = _NC * _NS
_IDX_TOTAL = B * N0
_IDX_PER_W = _IDX_TOTAL // _NW
_GATHERS_PER_W = _IDX_PER_W // 128
_HALF_G = _GATHERS_PER_W // 2
_ROWS_PER_ROUND = _HALF_G * 128
_OUT_PER_W = B // _NW


TNUM = 1000000
_CHUNK = 128
_NFULL_EVEN = 7808
_CPW = _NFULL_EVEN // _NW
_TAIL_E = TNUM - _NFULL_EVEN * _CHUNK
_PAIRS = TNUM // 2


_NBUF = 4


def _sc_transpose_body(tblT_hbm, tail_hbm, out_hbm, in_v, out_v, tail_v,
                       *sems):
  wid = lax.axis_index("s") * _NC + lax.axis_index("c")
  base = wid * _CPW
  sem_in = sems[:_NBUF]
  sem_out = sems[_NBUF:]

  def in_src(c):
    e0 = pl.multiple_of(c * _CHUNK, _CHUNK)
    return tblT_hbm.at[:, pl.ds(e0, _CHUNK)]

  def out_dst(c):
    r0 = pl.multiple_of(c * (_CHUNK // 2), _CHUNK // 2)
    return out_hbm.at[pl.ds(r0, _CHUNK // 2)]

  iota = lax.iota(jnp.int32, 16)
  tvecs = [iota + 16 * a for a in range(4)]
  em = [jnp.bitwise_and(iota + m, 15) for m in range(16)]
  srow = [lax.shift_right_logical(e, 1) for e in em]
  scolb = [jnp.bitwise_or(lax.shift_left(jnp.bitwise_and(e, 1), 6), iota)
           for e in em]

  for b in range(_NBUF):
    pltpu.async_copy(in_src(base + b), in_v.at[b], sem_in[b])

  def loop(nb, _):
    for b in range(_NBUF):
      c = base + _NBUF * nb + b
      pltpu.make_async_copy(in_src(c), in_v.at[b], sem_in[b]).wait()

      @pl.when(nb > 0)
      def _():
        pltpu.make_async_copy(out_v.at[b], out_dst(c), sem_out[b]).wait()

      src = in_v.at[b]
      dst = out_v.at[b]

      def eb_body(eb, _, src=src, dst=dst):
        for a in range(4):
          vals = []
          for m in range(16):
            ev = em[m] + 16 * eb
            vals.append(plsc.load_gather(src, [tvecs[a], ev]))
          for m in range(16):
            rv = srow[m] + 8 * eb
            cv = scolb[m] + 16 * a
            plsc.store_scatter(dst, [rv, cv], vals[m])
        return 0

      lax.fori_loop(0, 8, eb_body, 0)

      pltpu.async_copy(out_v.at[b], out_dst(c), sem_out[b])

      @pl.when(nb < _CPW // _NBUF - 1)
      def _():
        pltpu.async_copy(in_src(c + _NBUF), in_v.at[b], sem_in[b])
    return 0

  lax.fori_loop(0, _CPW // _NBUF, loop, 0)
  for b in range(_NBUF):
    pltpu.make_async_copy(out_v.at[b], out_dst(base), sem_out[b]).wait()

  @pl.when(wid == 0)
  def _():
    for t in range(_TAIL_E // 2 // 32):
      pltpu.sync_copy(tail_hbm.at[pl.ds(32 * t, 32)], tail_v)
      pltpu.sync_copy(
          tail_v,
          out_hbm.at[pl.ds(_NFULL_EVEN * (_CHUNK // 2) + 32 * t, 32)])


def _sc_pack(tableT, tail_pairs):
  mesh = plsc.VectorSubcoreMesh(core_axis_name="c", subcore_axis_name="s")
  return pl.kernel(
      _sc_transpose_body,
      out_type=jax.ShapeDtypeStruct((_PAIRS, 128), jnp.float32),
      mesh=mesh,
      scratch_types=[
          pltpu.VMEM((_NBUF, TDIM, _CHUNK), jnp.float32),
          pltpu.VMEM((_NBUF, _CHUNK // 2, 128), jnp.float32),
          pltpu.VMEM((32, 128), jnp.float32),
      ] + [pltpu.SemaphoreType.DMA] * (2 * _NBUF),
      compiler_params=pltpu.CompilerParams(needs_layout_passes=False),
  )(tableT, tail_pairs)


def _sc_gather_body(table_hbm, idx_hbm, out_hbm, idx_v, idxh_v, rows_v,
                    out_v, sem):
  wid = lax.axis_index("s") * _NC + lax.axis_index("c")
  pltpu.sync_copy(idx_hbm.at[wid], idx_v)
  for j in range(_GATHERS_PER_W):
    for c in range(128 // 16):
      sl = pl.ds(c * 16, 16)
      idxh_v[j, sl] = lax.shift_right_logical(idx_v[j, sl], 1)

  for half in range(2):
    copies = [
        pltpu.async_copy(table_hbm.at[idxh_v.at[half * _HALF_G + j]],
                         rows_v.at[pl.ds(j * 128, 128)], sem)
        for j in range(_HALF_G)
    ]
    for cp in copies:
      cp.wait()

    def group_mean(g, _, half=half):
      base = g * N0
      accs = [jnp.zeros((16,), jnp.float32) for _ in range(TDIM // 16)]
      for r in range(N0):
        row = base + r
        rv = jnp.full((16,), row, jnp.int32)
        jv = lax.shift_right_logical(rv, 7)
        kv = jnp.bitwise_and(rv, 127)
        orig = plsc.load_gather(
            idx_v, [jv + jnp.int32(half * _HALF_G), kv])
        odd = jnp.bitwise_and(orig, 1) == 1
        for c in range(TDIM // 16):
          lo = rows_v[row, pl.ds(c * 16, 16)]
          hi = rows_v[row, pl.ds(TDIM + c * 16, 16)]
          accs[c] = accs[c] + jnp.where(odd, hi, lo)
      for c in range(TDIM // 16):
        out_v[half * (_OUT_PER_W // 2) + g, pl.ds(c * 16, 16)] = (
            accs[c] * (1.0 / N0))
      return 0

    lax.fori_loop(0, _ROWS_PER_ROUND // N0, group_mean, 0)

  pltpu.sync_copy(out_v, out_hbm.at[pl.ds(wid * _OUT_PER_W, _OUT_PER_W)])


def _sc_gather_mean(pair_table, idx3d):
  mesh = plsc.VectorSubcoreMesh(core_axis_name="c", subcore_axis_name="s")
  return pl.kernel(
      _sc_gather_body,
      out_type=jax.ShapeDtypeStruct((B, TDIM), jnp.float32),
      mesh=mesh,
      scratch_types=[
          pltpu.VMEM((_GATHERS_PER_W, 128), jnp.int32),
          pltpu.VMEM((_GATHERS_PER_W, 128), jnp.int32),
          pltpu.VMEM((_ROWS_PER_ROUND, 128), jnp.float32),
          pltpu.VMEM((_OUT_PER_W, TDIM), jnp.float32),
          pltpu.SemaphoreType.DMA,
      ],
      compiler_params=pltpu.CompilerParams(needs_layout_passes=False),
  )(pair_table, idx3d)


_BB = 128


def _tca_body(x0_ref, x1_ref, x2_ref, xb2_ref, w_ref, a10_ref, a50_ref,
              e0_ref, e1_ref, e2_ref, b2_ref):
  w = w_ref[...]
  e0_ref[...] = lax.dot(x0_ref[...], w, preferred_element_type=jnp.float32)
  y1 = lax.dot(x1_ref[...], w, preferred_element_type=jnp.float32)
  e1_ref[...] = lax.dot(a10_ref[...], y1,
                        preferred_element_type=jnp.float32) * (1.0 / N0)
  y2 = lax.dot(x2_ref[...], w, preferred_element_type=jnp.float32)
  e2_ref[...] = lax.dot(a50_ref[...], y2,
                        preferred_element_type=jnp.float32) * (1.0 / (N0 * N1))
  yb = lax.dot(xb2_ref[...], w, preferred_element_type=jnp.float32)
  b2_ref[...] = lax.dot(a50_ref[...], yb,
                        preferred_element_type=jnp.float32) * (1.0 / (N0 * N1))


def _tc_a(x_0, x_1, x_2, x_bi_2, weight_trans, a10, a50):
  grid = (B // _BB,)
  out = jax.ShapeDtypeStruct((B, TDIM), jnp.float32)
  return pl.pallas_call(
      _tca_body,
      grid=grid,
      in_specs=[
          pl.BlockSpec((_BB, NFEAT), lambda i: (i, 0)),
          pl.BlockSpec((_BB * N0, NFEAT), lambda i: (i, 0)),
          pl.BlockSpec((_BB * N0 * N1, NFEAT), lambda i: (i, 0)),
          pl.BlockSpec((_BB * N0 * N1, NFEAT), lambda i: (i, 0)),
          pl.BlockSpec((NFEAT, TDIM), lambda i: (0, 0)),
          pl.BlockSpec((_BB, _BB * N0), lambda i: (0, 0)),
          pl.BlockSpec((_BB, _BB * N0 * N1), lambda i: (0, 0)),
      ],
      out_specs=[pl.BlockSpec((_BB, TDIM), lambda i: (i, 0))] * 4,
      out_shape=[out, out, out, out],
  )(x_0, x_1, x_2, x_bi_2, weight_trans, a10, a50)


_BBB = 1024


def _tcb_body(e0_ref, e1_ref, e2_ref, b2_ref, b1_ref, w1_ref, l2wt_ref,
              l2b_ref, out_ref):
  e0, e1, e2 = e0_ref[...], e1_ref[...], e2_ref[...]
  b1, b2 = b1_ref[...], b2_ref[...]
  terms = [e0 * b1, e0 * b2, e1 * b1, e1 * b2, e2 * b1, e2 * b2,
           e0 * e1, e0 * e2, e1 * e2, e0, e1, e2]
  w1 = w1_ref[...]
  logits = [jnp.sum(t * w1, axis=1, keepdims=True) for t in terms]
  mx = logits[0]
  for l in logits[1:]:
    mx = jnp.maximum(mx, l)
  exps = [jnp.exp(l - mx) for l in logits]
  denom = exps[0]
  for e in exps[1:]:
    denom = denom + e
  hidden = exps[0] * terms[0]
  for e, t in zip(exps[1:], terms[1:]):
    hidden = hidden + e * t
  hidden = hidden / denom

  out = lax.dot(hidden, l2wt_ref[...],
                preferred_element_type=jnp.float32) + l2b_ref[...]
  col = lax.broadcasted_iota(jnp.int32, out.shape, 1)
  out = jnp.where(col < NCLASS, out, -1e30)
  omx = jnp.max(out, axis=1, keepdims=True)
  sh = out - omx
  lse = jnp.log(jnp.sum(jnp.exp(sh), axis=1, keepdims=True))
  out_ref[...] = sh - lse


def _tc_b(e0, e1, e2, b2, b1, lin1_w, l2wt, l2b):
  grid = (B // _BBB,)
  return pl.pallas_call(
      _tcb_body,
      grid=grid,
      in_specs=[pl.BlockSpec((_BBB, TDIM), lambda i: (i, 0))] * 5 + [
          pl.BlockSpec((1, TDIM), lambda i: (0, 0)),
          pl.BlockSpec((TDIM, NCPAD), lambda i: (0, 0)),
          pl.BlockSpec((1, NCPAD), lambda i: (0, 0)),
      ],
      out_specs=pl.BlockSpec((_BBB, NCPAD), lambda i: (i, 0)),
      out_shape=jax.ShapeDtypeStruct((B, NCPAD), jnp.float32),
  )(e0, e1, e2, b2, b1, lin1_w, l2wt, l2b)


def kernel(x_0, x_1, x_2, x_bi_0, x_bi_1, x_bi_2, weight_trans, table,
           lin1_w, lin1_b, lin2_w, lin2_b):
  del x_bi_0, lin1_b
  ri10 = lax.broadcasted_iota(jnp.int32, (_BB, _BB * N0), 1) // N0
  a10 = (ri10 == lax.broadcasted_iota(jnp.int32, (_BB, _BB * N0), 0)
         ).astype(jnp.float32)
  ri50 = lax.broadcasted_iota(jnp.int32, (_BB, _BB * N0 * N1), 1) // (N0 * N1)
  a50 = (ri50 == lax.broadcasted_iota(jnp.int32, (_BB, _BB * N0 * N1), 0)
         ).astype(jnp.float32)

  e0, e1, e2, b2 = _tc_a(x_0, x_1, x_2, x_bi_2, weight_trans, a10, a50)

  idx3d = x_bi_1.astype(jnp.int32).reshape(_NW, _GATHERS_PER_W, 128)
  tail_pairs = table[_NFULL_EVEN * _CHUNK:].reshape(_TAIL_E // 2, 128)
  pair_table = _sc_pack(table.T, tail_pairs)
  b1 = _sc_gather_mean(pair_table, idx3d)

  l2wt = jnp.zeros((TDIM, NCPAD), jnp.float32).at[:, :NCLASS].set(lin2_w.T)
  l2b = jnp.zeros((1, NCPAD), jnp.float32).at[:, :NCLASS].set(lin2_b)
  out = _tc_b(e0, e1, e2, b2, b1, lin1_w, l2wt, l2b)
  return out[:, :NCLASS]
